# baseline (speedup 1.0000x reference)
import functools

import jax
import jax.numpy as jnp
from jax import lax
from jax.experimental import pallas as pl
from jax.experimental.pallas import tpu as pltpu
from jax.experimental.pallas import tpu_sc as plsc

T = 128
LEAK = 0.01
NG = 64
NSUB = 16
ECH = 128


def _leaky(x):
    return jnp.where(x >= 0, x, LEAK * x)


def _make_sc_segsum(NH, P, CH, with_xgather, XC=0, XCH=0):
    SR = P // NSUB
    mesh = plsc.VectorSubcoreMesh(core_axis_name="c", subcore_axis_name="s")
    out_type = [jax.ShapeDtypeStruct((2, P, 128), jnp.float32)]
    if with_xgather:
        out_type.append(jax.ShapeDtypeStruct((2, P, 128), jnp.float32))
    scratch = [
        pltpu.VMEM((4, 2, ECH), jnp.int32),
        pltpu.VMEM((2, ECH, 128), jnp.float32),
        pltpu.VMEM_SHARED((P, 128), jnp.float32),
        pltpu.SemaphoreType.DMA,
        pltpu.SemaphoreType.DMA,
        pltpu.SemaphoreType.DMA,
        pltpu.SemaphoreType.DMA,
        pltpu.SemaphoreType.DMA,
        pltpu.SemaphoreType.DMA,
    ]
    if with_xgather:
        scratch.append(pltpu.VMEM((XC,), jnp.int32))

    def body(h_hbm, idx_hbm, zeros_hbm, *rest):
        if with_xgather:
            (nos_hbm, agg_hbm, xp_hbm, idxb, rows, acc,
             is0, is1, is2, is3, gs0, gs1, nidx) = rest
        else:
            agg_hbm, idxb, rows, acc, is0, is1, is2, is3, gs0, gs1 = rest
        isems = (is0, is1, is2, is3)
        gsems = (gs0, gs1)
        c = lax.axis_index("c")
        s = lax.axis_index("s")
        pltpu.sync_copy(zeros_hbm.at[pl.ds(s * SR, SR)],
                        acc.at[pl.ds(s * SR, SR)])

        def idx_op(g, j):
            return pltpu.make_async_copy(idx_hbm.at[s, g], idxb.at[j],
                                         isems[j])

        def gather_op(j, b):
            return pltpu.make_async_copy(
                h_hbm.at[c].at[idxb.at[j, 0]], rows.at[b], gsems[b])

        for j in range(4):
            idx_op(j, j).start()
        for j in range(2):
            idx_op(j, j).wait()
            gather_op(j, j).start()
        plsc.subcore_barrier()

        def outer(i, carry):
            g0 = 4 * i
            for j in range(4):
                g = g0 + j
                b = j % 2

                @pl.when(g < CH)
                def _():
                    gather_op(j, b).wait()
                    pltpu.sync_copy(rows.at[b], acc.at[idxb.at[j, 1]],
                                    add=True)

                    @pl.when(g + 4 < CH)
                    def _():
                        idx_op(g + 4, j).start()

                    @pl.when(g + 2 < CH)
                    def _():
                        idx_op(g + 2, (j + 2) % 4).wait()
                        gather_op((j + 2) % 4, b).start()
            return carry

        lax.fori_loop(0, (CH + 3) // 4, outer, 0)
        plsc.subcore_barrier()
        pltpu.sync_copy(acc.at[pl.ds(s * SR, SR)],
                        agg_hbm.at[c, pl.ds(s * SR, SR)])
        if with_xgather:
            for k in range(XCH):
                pltpu.sync_copy(nos_hbm.at[s, k], nidx)
                pltpu.async_copy(h_hbm.at[c].at[nidx],
                                 rows.at[0].at[pl.ds(0, XC)], gs0).wait()
                pltpu.sync_copy(rows.at[0].at[pl.ds(0, XC)],
                                xp_hbm.at[c, pl.ds(s * SR + k * XC, XC)])

    return pl.kernel(body, out_type=out_type, mesh=mesh,
                     scratch_types=scratch)


def _grouped_conv(P, NT, NB, tile_deg, tile_valid, A, H, Wl_, Wr_, bl_):
    D = Wl_.shape[1]
    HD = D // 2

    def body(td_ref, tv_ref, a_ref, h_ref, wl_ref, wr_ref, b_ref, o_ref):
        i = pl.program_id(0)
        a = jnp.concatenate([a_ref[0], a_ref[1]], axis=1)
        hh = jnp.concatenate([h_ref[0], h_ref[1]], axis=1)
        out = (jnp.dot(a, wl_ref[0], preferred_element_type=jnp.float32)
               + jnp.dot(hh, wr_ref[0], preferred_element_type=jnp.float32))
        d = td_ref[i]
        bias = jnp.zeros((D,), jnp.float32)
        for dd in range(NB):
            bias = jnp.where(d == dd, b_ref[dd], bias)
        out = _leaky(out + bias[None, :])
        rid = lax.broadcasted_iota(jnp.int32, (T, 1), 0)
        out = jnp.where(rid < tv_ref[i], out, 0.0)
        o_ref[0] = out[:, :HD]
        o_ref[1] = out[:, HD:]

    grid_spec = pltpu.PrefetchScalarGridSpec(
        num_scalar_prefetch=2,
        grid=(NT,),
        in_specs=[
            pl.BlockSpec((2, T, HD), lambda i, td, tv: (0, i, 0)),
            pl.BlockSpec((2, T, HD), lambda i, td, tv: (0, i, 0)),
            pl.BlockSpec((1, D, D), lambda i, td, tv: (td[i], 0, 0)),
            pl.BlockSpec((1, D, D), lambda i, td, tv: (td[i], 0, 0)),
            pl.BlockSpec((NB, D), lambda i, td, tv: (0, 0)),
        ],
        out_specs=pl.BlockSpec((2, T, HD), lambda i, td, tv: (0, i, 0)),
    )
    return pl.pallas_call(
        body, grid_spec=grid_spec,
        out_shape=jax.ShapeDtypeStruct((2, P, HD), jnp.float32),
    )(tile_deg, tile_valid, A, H, Wl_, Wr_, bl_)


def _pool(P, NT, H, batch3d):
    def body(h_ref, b_ref, o_ref):
        i = pl.program_id(0)

        @pl.when(i == 0)
        def _():
            o_ref[...] = jnp.zeros_like(o_ref)

        bt = b_ref[0, 0]
        gid = lax.broadcasted_iota(jnp.int32, (NG, T), 0)
        oh = (gid == bt[None, :]).astype(jnp.float32)
        hh = jnp.concatenate([h_ref[0], h_ref[1]], axis=1)
        o_ref[...] += jnp.dot(oh, hh, preferred_element_type=jnp.float32)

    return pl.pallas_call(
        body,
        grid=(NT,),
        in_specs=[
            pl.BlockSpec((2, T, 128), lambda i: (0, i, 0)),
            pl.BlockSpec((1, 1, T), lambda i: (i, 0, 0)),
        ],
        out_specs=pl.BlockSpec((NG, 256), lambda i: (0, 0)),
        out_shape=jax.ShapeDtypeStruct((NG, 256), jnp.float32),
    )(H, batch3d)


def _classifier(P, NT, NC, h1, h2, h3, batch3d, pool, C1_W, C1_b2, CW, Cb,
                F_Wp, F_bp):
    def body(h1_ref, h2_ref, h3_ref, b_ref, p_ref, c1w_ref, c1b_ref,
             cw_ref, cb_ref, fw_ref, fb_ref, o_ref):
        bt = b_ref[0, 0]
        gid = lax.broadcasted_iota(jnp.int32, (T, NG), 1)
        oh = (gid == bt[:, None]).astype(jnp.float32)
        hp = jnp.dot(oh, p_ref[...], preferred_element_type=jnp.float32)
        hcat = jnp.concatenate(
            [jnp.concatenate([h1_ref[0], h1_ref[1]], axis=1),
             jnp.concatenate([h2_ref[0], h2_ref[1]], axis=1),
             jnp.concatenate([h3_ref[0], h3_ref[1]], axis=1),
             hp], axis=1)
        z = jnp.dot(hcat, c1w_ref[...],
                    preferred_element_type=jnp.float32) + c1b_ref[...]
        for l in range(NC):
            z = _leaky(jnp.dot(z, cw_ref[l],
                               preferred_element_type=jnp.float32)
                       + cb_ref[l][None, :])
        y = jnp.dot(z, fw_ref[...], preferred_element_type=jnp.float32)
        o_ref[...] = jax.nn.sigmoid(y + fb_ref[...])

    DC = C1_W.shape[1]
    return pl.pallas_call(
        body,
        grid=(NT,),
        in_specs=[
            pl.BlockSpec((2, T, 128), lambda i: (0, i, 0)),
            pl.BlockSpec((2, T, 128), lambda i: (0, i, 0)),
            pl.BlockSpec((2, T, 128), lambda i: (0, i, 0)),
            pl.BlockSpec((1, 1, T), lambda i: (i, 0, 0)),
            pl.BlockSpec((NG, 256), lambda i: (0, 0)),
            pl.BlockSpec(C1_W.shape, lambda i: (0, 0)),
            pl.BlockSpec((1, DC), lambda i: (0, 0)),
            pl.BlockSpec(CW.shape, lambda i: (0, 0, 0)),
            pl.BlockSpec(Cb.shape, lambda i: (0, 0)),
            pl.BlockSpec(F_Wp.shape, lambda i: (0, 0)),
            pl.BlockSpec((1, 128), lambda i: (0, 0)),
        ],
        out_specs=pl.BlockSpec((T, 128), lambda i: (i, 0)),
        out_shape=jax.ShapeDtypeStruct((P, 128), jnp.float32),
    )(h1, h2, h3, batch3d, pool, C1_W, C1_b2, CW, Cb, F_Wp, F_bp)


def _routing(deg, NB, N, P, NT):
    counts = jnp.bincount(deg, length=NB).astype(jnp.int32)
    padded = ((counts + T - 1) // T) * T
    z1 = jnp.zeros((1,), jnp.int32)
    pstart = jnp.concatenate([z1, jnp.cumsum(padded)])[:NB]
    sstart = jnp.concatenate([z1, jnp.cumsum(counts)])[:NB]
    perm = jnp.argsort(deg, stable=True).astype(jnp.int32)

    slots = jnp.arange(P, dtype=jnp.int32)
    b = (jnp.searchsorted(pstart, slots, side="right") - 1).astype(jnp.int32)
    off = slots - pstart[b]
    valid = off < counts[b]
    node_of_slot = jnp.where(
        valid, perm[jnp.clip(sstart[b] + off, 0, N - 1)], 0).astype(jnp.int32)
    scat = jnp.where(valid, node_of_slot, N)
    slot_of_node = jnp.zeros((N,), jnp.int32).at[scat].set(slots, mode="drop")

    tstart = jnp.arange(NT, dtype=jnp.int32) * T
    tb = (jnp.searchsorted(pstart, tstart, side="right") - 1).astype(jnp.int32)
    tile_deg = tb
    tile_valid = jnp.clip(counts[tb] - (tstart - pstart[tb]), 0, T).astype(
        jnp.int32)
    return node_of_slot, slot_of_node, tile_deg, tile_valid


def kernel(x, edge_index, edge_attr, batch, Wl1, bl1, Wr1, Wl, bl, Wr,
           C1_W, C1_b, CW, Cb, F_W, F_b):
    N, D = x.shape
    E = edge_index.shape[1]
    NB = Wl1.shape[0]
    NCONV = Wl.shape[0]
    NC = CW.shape[0]
    DC = C1_W.shape[1]

    NT = -(-(N + NB * (T - 1)) // T)
    P = NT * T
    src = edge_index[0]
    dst = edge_index[1]

    deg = jnp.minimum(jnp.bincount(dst, length=N), NB - 1).astype(jnp.int32)
    node_of_slot, slot_of_node, tile_deg, tile_valid = _routing(
        deg, NB, N, P, NT)

    EP = -(-E // (NSUB * ECH)) * (NSUB * ECH)
    CH = EP // (NSUB * ECH)
    padn = EP - E
    dst_slot = slot_of_node[dst]
    src_slot = slot_of_node[src]
    trash = jnp.full((padn,), P - 1, jnp.int32)
    src_l1 = jnp.concatenate([src.astype(jnp.int32),
                              jnp.zeros((padn,), jnp.int32)]).reshape(
                                  NSUB, CH, ECH)
    src_ln = jnp.concatenate([src_slot, trash]).reshape(NSUB, CH, ECH)
    dst_e = jnp.concatenate([dst_slot, trash]).reshape(NSUB, CH, ECH)
    idx_l1 = jnp.stack([src_l1, dst_e], axis=2)
    idx_ln = jnp.stack([src_ln, dst_e], axis=2)

    SR = P // NSUB
    XC = 120
    XCH = SR // XC
    nos3 = node_of_slot.reshape(NSUB, XCH, XC)

    batch3d = batch[node_of_slot].astype(jnp.int32).reshape(NT, 1, T)
    x_split = jnp.stack([x[:, :128], x[:, 128:]])
    zerosP = jnp.zeros((P, 128), jnp.float32)

    sc1 = _make_sc_segsum(N, P, CH, True, XC, XCH)
    scn = _make_sc_segsum(P, P, CH, False)

    agg1, xperm = sc1(x_split, idx_l1, zerosP, nos3)
    h = _grouped_conv(P, NT, NB, tile_deg, tile_valid, agg1, xperm,
                      Wl1, Wr1, bl1)
    hs = []
    for l in range(NCONV):
        (aggl,) = scn(h, idx_ln, zerosP)
        h = _grouped_conv(P, NT, NB, tile_deg, tile_valid, aggl, h,
                          Wl[l], Wr[l], bl[l])
        hs.append(h)

    pool = _pool(P, NT, hs[-1], batch3d)

    F_Wp = jnp.pad(F_W, ((0, 0), (0, 127)))
    F_bp = jnp.pad(F_b.reshape(1, 1), ((0, 0), (0, 127)))
    cls = _classifier(P, NT, NC, hs[0], hs[1], hs[2], batch3d, pool,
                      C1_W, C1_b.reshape(1, DC), CW, Cb, F_Wp, F_bp)
    return cls[slot_of_node, 0][:, None]


# speedup vs baseline: 1.0112x; 1.0112x over previous
import functools

import jax
import jax.numpy as jnp
from jax import lax
from jax.experimental import pallas as pl
from jax.experimental.pallas import tpu as pltpu
from jax.experimental.pallas import tpu_sc as plsc

T = 128
LEAK = 0.01
NG = 64
NSUB = 16
ECH = 128


def _leaky(x):
    return jnp.where(x >= 0, x, LEAK * x)


def _make_sc_segsum(NH, P, CH, with_xgather, XC=0, XCH=0):
    SR = P // NSUB
    mesh = plsc.VectorSubcoreMesh(core_axis_name="c", subcore_axis_name="s")
    out_type = [jax.ShapeDtypeStruct((2, P, 128), jnp.float32)]
    if with_xgather:
        out_type.append(jax.ShapeDtypeStruct((2, P, 128), jnp.float32))
    scratch = [
        pltpu.VMEM((4, 2, ECH), jnp.int32),
        pltpu.VMEM((2, ECH, 128), jnp.float32),
        pltpu.VMEM_SHARED((P, 128), jnp.float32),
        pltpu.SemaphoreType.DMA,
        pltpu.SemaphoreType.DMA,
        pltpu.SemaphoreType.DMA,
        pltpu.SemaphoreType.DMA,
        pltpu.SemaphoreType.DMA,
        pltpu.SemaphoreType.DMA,
    ]
    if with_xgather:
        scratch.append(pltpu.VMEM((XC,), jnp.int32))

    def body(h_hbm, idx_hbm, zeros_hbm, *rest):
        if with_xgather:
            (nos_hbm, agg_hbm, xp_hbm, idxb, rows, acc,
             is0, is1, is2, is3, gs0, gs1, nidx) = rest
        else:
            agg_hbm, idxb, rows, acc, is0, is1, is2, is3, gs0, gs1 = rest
        isems = (is0, is1, is2, is3)
        gsems = (gs0, gs1)
        c = lax.axis_index("c")
        s = lax.axis_index("s")
        pltpu.sync_copy(zeros_hbm.at[pl.ds(s * SR, SR)],
                        acc.at[pl.ds(s * SR, SR)])

        def idx_op(g, j):
            return pltpu.make_async_copy(idx_hbm.at[s, g], idxb.at[j],
                                         isems[j])

        def gather_op(j, b):
            return pltpu.make_async_copy(
                h_hbm.at[c].at[idxb.at[j, 0]], rows.at[b], gsems[b])

        for j in range(4):
            idx_op(j, j).start()
        for j in range(2):
            idx_op(j, j).wait()
            gather_op(j, j).start()
        plsc.subcore_barrier()

        def outer(i, carry):
            g0 = 4 * i
            for j in range(4):
                g = g0 + j
                b = j % 2

                @pl.when(g < CH)
                def _():
                    gather_op(j, b).wait()
                    pltpu.sync_copy(rows.at[b], acc.at[idxb.at[j, 1]],
                                    add=True)

                    @pl.when(g + 4 < CH)
                    def _():
                        idx_op(g + 4, j).start()

                    @pl.when(g + 2 < CH)
                    def _():
                        idx_op(g + 2, (j + 2) % 4).wait()
                        gather_op((j + 2) % 4, b).start()
            return carry

        lax.fori_loop(0, (CH + 3) // 4, outer, 0)
        plsc.subcore_barrier()
        pltpu.sync_copy(acc.at[pl.ds(s * SR, SR)],
                        agg_hbm.at[c, pl.ds(s * SR, SR)])
        if with_xgather:
            for k in range(XCH):
                pltpu.sync_copy(nos_hbm.at[s, k], nidx)
                pltpu.async_copy(h_hbm.at[c].at[nidx],
                                 rows.at[0].at[pl.ds(0, XC)], gs0).wait()
                pltpu.sync_copy(rows.at[0].at[pl.ds(0, XC)],
                                xp_hbm.at[c, pl.ds(s * SR + k * XC, XC)])

    return pl.kernel(body, out_type=out_type, mesh=mesh,
                     scratch_types=scratch)


def _grouped_conv(P, NT, NB, tile_deg, tile_valid, A, H, Wl_, Wr_, bl_):
    D = Wl_.shape[1]
    HD = D // 2

    def body(td_ref, tv_ref, a_ref, h_ref, wl_ref, wr_ref, b_ref, o_ref):
        i = pl.program_id(0)
        a = jnp.concatenate([a_ref[0], a_ref[1]], axis=1)
        hh = jnp.concatenate([h_ref[0], h_ref[1]], axis=1)
        out = (jnp.dot(a, wl_ref[0], preferred_element_type=jnp.float32)
               + jnp.dot(hh, wr_ref[0], preferred_element_type=jnp.float32))
        d = td_ref[i]
        bias = jnp.zeros((D,), jnp.float32)
        for dd in range(NB):
            bias = jnp.where(d == dd, b_ref[dd], bias)
        out = _leaky(out + bias[None, :])
        rid = lax.broadcasted_iota(jnp.int32, (T, 1), 0)
        out = jnp.where(rid < tv_ref[i], out, 0.0)
        o_ref[0] = out[:, :HD]
        o_ref[1] = out[:, HD:]

    grid_spec = pltpu.PrefetchScalarGridSpec(
        num_scalar_prefetch=2,
        grid=(NT,),
        in_specs=[
            pl.BlockSpec((2, T, HD), lambda i, td, tv: (0, i, 0)),
            pl.BlockSpec((2, T, HD), lambda i, td, tv: (0, i, 0)),
            pl.BlockSpec((1, D, D), lambda i, td, tv: (td[i], 0, 0)),
            pl.BlockSpec((1, D, D), lambda i, td, tv: (td[i], 0, 0)),
            pl.BlockSpec((NB, D), lambda i, td, tv: (0, 0)),
        ],
        out_specs=pl.BlockSpec((2, T, HD), lambda i, td, tv: (0, i, 0)),
    )
    return pl.pallas_call(
        body, grid_spec=grid_spec,
        out_shape=jax.ShapeDtypeStruct((2, P, HD), jnp.float32),
    )(tile_deg, tile_valid, A, H, Wl_, Wr_, bl_)


def _pool(P, NT, H, batch3d):
    def body(h_ref, b_ref, o_ref):
        i = pl.program_id(0)

        @pl.when(i == 0)
        def _():
            o_ref[...] = jnp.zeros_like(o_ref)

        bt = b_ref[0, 0]
        gid = lax.broadcasted_iota(jnp.int32, (NG, T), 0)
        oh = (gid == bt[None, :]).astype(jnp.float32)
        hh = jnp.concatenate([h_ref[0], h_ref[1]], axis=1)
        o_ref[...] += jnp.dot(oh, hh, preferred_element_type=jnp.float32)

    return pl.pallas_call(
        body,
        grid=(NT,),
        in_specs=[
            pl.BlockSpec((2, T, 128), lambda i: (0, i, 0)),
            pl.BlockSpec((1, 1, T), lambda i: (i, 0, 0)),
        ],
        out_specs=pl.BlockSpec((NG, 256), lambda i: (0, 0)),
        out_shape=jax.ShapeDtypeStruct((NG, 256), jnp.float32),
    )(H, batch3d)


def _classifier(P, NT, NC, h1, h2, h3, batch3d, pool, C1_W, C1_b2, CW, Cb,
                F_Wp, F_bp):
    def body(h1_ref, h2_ref, h3_ref, b_ref, p_ref, c1w_ref, c1b_ref,
             cw_ref, cb_ref, fw_ref, fb_ref, o_ref):
        bt = b_ref[0, 0]
        gid = lax.broadcasted_iota(jnp.int32, (T, NG), 1)
        oh = (gid == bt[:, None]).astype(jnp.float32)
        hp = jnp.dot(oh, p_ref[...], preferred_element_type=jnp.float32)
        hcat = jnp.concatenate(
            [jnp.concatenate([h1_ref[0], h1_ref[1]], axis=1),
             jnp.concatenate([h2_ref[0], h2_ref[1]], axis=1),
             jnp.concatenate([h3_ref[0], h3_ref[1]], axis=1),
             hp], axis=1)
        z = jnp.dot(hcat, c1w_ref[...],
                    preferred_element_type=jnp.float32) + c1b_ref[...]
        for l in range(NC):
            z = _leaky(jnp.dot(z, cw_ref[l],
                               preferred_element_type=jnp.float32)
                       + cb_ref[l][None, :])
        y = jnp.dot(z, fw_ref[...], preferred_element_type=jnp.float32)
        o_ref[...] = jax.nn.sigmoid(y + fb_ref[...])

    DC = C1_W.shape[1]
    return pl.pallas_call(
        body,
        grid=(NT,),
        in_specs=[
            pl.BlockSpec((2, T, 128), lambda i: (0, i, 0)),
            pl.BlockSpec((2, T, 128), lambda i: (0, i, 0)),
            pl.BlockSpec((2, T, 128), lambda i: (0, i, 0)),
            pl.BlockSpec((1, 1, T), lambda i: (i, 0, 0)),
            pl.BlockSpec((NG, 256), lambda i: (0, 0)),
            pl.BlockSpec(C1_W.shape, lambda i: (0, 0)),
            pl.BlockSpec((1, DC), lambda i: (0, 0)),
            pl.BlockSpec(CW.shape, lambda i: (0, 0, 0)),
            pl.BlockSpec(Cb.shape, lambda i: (0, 0)),
            pl.BlockSpec(F_Wp.shape, lambda i: (0, 0)),
            pl.BlockSpec((1, 128), lambda i: (0, 0)),
        ],
        out_specs=pl.BlockSpec((T, 128), lambda i: (i, 0)),
        out_shape=jax.ShapeDtypeStruct((P, 128), jnp.float32),
    )(h1, h2, h3, batch3d, pool, C1_W, C1_b2, CW, Cb, F_Wp, F_bp)


def _routing(deg, NB, N, P, NT):
    counts = jnp.bincount(deg, length=NB).astype(jnp.int32)
    padded = ((counts + T - 1) // T) * T
    z1 = jnp.zeros((1,), jnp.int32)
    pstart = jnp.concatenate([z1, jnp.cumsum(padded)])[:NB]
    sstart = jnp.concatenate([z1, jnp.cumsum(counts)])[:NB]
    perm = jnp.argsort(deg, stable=True).astype(jnp.int32)

    slots = jnp.arange(P, dtype=jnp.int32)
    b = (jnp.searchsorted(pstart, slots, side="right") - 1).astype(jnp.int32)
    off = slots - pstart[b]
    valid = off < counts[b]
    node_of_slot = jnp.where(
        valid,
        lax.optimization_barrier(perm[jnp.clip(sstart[b] + off, 0, N - 1)]),
        0).astype(jnp.int32)
    scat = jnp.where(valid, node_of_slot, N)
    slot_of_node = jnp.zeros((N,), jnp.int32).at[scat].set(slots, mode="drop")

    tstart = jnp.arange(NT, dtype=jnp.int32) * T
    tb = (jnp.searchsorted(pstart, tstart, side="right") - 1).astype(jnp.int32)
    tile_deg = tb
    tile_valid = jnp.clip(counts[tb] - (tstart - pstart[tb]), 0, T).astype(
        jnp.int32)
    return node_of_slot, slot_of_node, tile_deg, tile_valid


def kernel(x, edge_index, edge_attr, batch, Wl1, bl1, Wr1, Wl, bl, Wr,
           C1_W, C1_b, CW, Cb, F_W, F_b):
    N, D = x.shape
    E = edge_index.shape[1]
    NB = Wl1.shape[0]
    NCONV = Wl.shape[0]
    NC = CW.shape[0]
    DC = C1_W.shape[1]

    NT = -(-(N + NB * (T - 1)) // T)
    P = NT * T
    src = edge_index[0]
    dst = edge_index[1]

    deg = jnp.minimum(jnp.bincount(dst, length=N), NB - 1).astype(jnp.int32)
    node_of_slot, slot_of_node, tile_deg, tile_valid = _routing(
        deg, NB, N, P, NT)

    EP = -(-E // (NSUB * ECH)) * (NSUB * ECH)
    CH = EP // (NSUB * ECH)
    padn = EP - E
    dst_slot = lax.optimization_barrier(slot_of_node[dst])
    src_slot = lax.optimization_barrier(slot_of_node[src])
    trash = jnp.full((padn,), P - 1, jnp.int32)
    src_l1 = jnp.concatenate([src.astype(jnp.int32),
                              jnp.zeros((padn,), jnp.int32)]).reshape(
                                  NSUB, CH, ECH)
    src_ln = jnp.concatenate([src_slot, trash]).reshape(NSUB, CH, ECH)
    dst_e = jnp.concatenate([dst_slot, trash]).reshape(NSUB, CH, ECH)
    idx_l1 = jnp.stack([src_l1, dst_e], axis=2)
    idx_ln = jnp.stack([src_ln, dst_e], axis=2)

    SR = P // NSUB
    XC = 120
    XCH = SR // XC
    nos3 = node_of_slot.reshape(NSUB, XCH, XC)

    batch3d = lax.optimization_barrier(
        batch[node_of_slot].astype(jnp.int32)).reshape(NT, 1, T)
    x_split = jnp.stack([x[:, :128], x[:, 128:]])
    zerosP = jnp.zeros((P, 128), jnp.float32)

    sc1 = _make_sc_segsum(N, P, CH, True, XC, XCH)
    scn = _make_sc_segsum(P, P, CH, False)

    agg1, xperm = sc1(x_split, idx_l1, zerosP, nos3)
    h = _grouped_conv(P, NT, NB, tile_deg, tile_valid, agg1, xperm,
                      Wl1, Wr1, bl1)
    hs = []
    for l in range(NCONV):
        (aggl,) = scn(h, idx_ln, zerosP)
        h = _grouped_conv(P, NT, NB, tile_deg, tile_valid, aggl, h,
                          Wl[l], Wr[l], bl[l])
        hs.append(h)

    pool = _pool(P, NT, hs[-1], batch3d)

    F_Wp = jnp.pad(F_W, ((0, 0), (0, 127)))
    F_bp = jnp.pad(F_b.reshape(1, 1), ((0, 0), (0, 127)))
    cls = _classifier(P, NT, NC, hs[0], hs[1], hs[2], batch3d, pool,
                      C1_W, C1_b.reshape(1, DC), CW, Cb, F_Wp, F_bp)
    return lax.optimization_barrier(cls[:, 0][slot_of_node])[:, None]


# speedup vs baseline: 2.3461x; 2.3201x over previous
import functools

import jax
import jax.numpy as jnp
from jax import lax
from jax.experimental import pallas as pl
from jax.experimental.pallas import tpu as pltpu
from jax.experimental.pallas import tpu_sc as plsc
from jax.experimental.compute_on import compute_on

T = 128
LEAK = 0.01
NG = 64
NSUB = 16
ECH = 128


def _leaky(x):
    return jnp.where(x >= 0, x, LEAK * x)


def _make_sc_segsum(NH, P, CH, with_xgather, XC=0, XCH=0):
    SR = P // NSUB
    mesh = plsc.VectorSubcoreMesh(core_axis_name="c", subcore_axis_name="s")
    out_type = [jax.ShapeDtypeStruct((2, P, 128), jnp.float32)]
    if with_xgather:
        out_type.append(jax.ShapeDtypeStruct((2, P, 128), jnp.float32))
    scratch = [
        pltpu.VMEM((4, 2, ECH), jnp.int32),
        pltpu.VMEM((2, ECH, 128), jnp.float32),
        pltpu.VMEM_SHARED((P, 128), jnp.float32),
        pltpu.SemaphoreType.DMA,
        pltpu.SemaphoreType.DMA,
        pltpu.SemaphoreType.DMA,
        pltpu.SemaphoreType.DMA,
        pltpu.SemaphoreType.DMA,
        pltpu.SemaphoreType.DMA,
    ]
    if with_xgather:
        scratch.append(pltpu.VMEM((XC,), jnp.int32))

    def body(h_hbm, idx_hbm, zeros_hbm, *rest):
        if with_xgather:
            (nos_hbm, agg_hbm, xp_hbm, idxb, rows, acc,
             is0, is1, is2, is3, gs0, gs1, nidx) = rest
        else:
            agg_hbm, idxb, rows, acc, is0, is1, is2, is3, gs0, gs1 = rest
        isems = (is0, is1, is2, is3)
        gsems = (gs0, gs1)
        c = lax.axis_index("c")
        s = lax.axis_index("s")
        pltpu.sync_copy(zeros_hbm.at[pl.ds(s * SR, SR)],
                        acc.at[pl.ds(s * SR, SR)])

        def idx_op(g, j):
            return pltpu.make_async_copy(idx_hbm.at[s, g], idxb.at[j],
                                         isems[j])

        def gather_op(j, b):
            return pltpu.make_async_copy(
                h_hbm.at[c].at[idxb.at[j, 0]], rows.at[b], gsems[b])

        for j in range(4):
            idx_op(j, j).start()
        for j in range(2):
            idx_op(j, j).wait()
            gather_op(j, j).start()
        plsc.subcore_barrier()

        def outer(i, carry):
            g0 = 4 * i
            for j in range(4):
                g = g0 + j
                b = j % 2

                @pl.when(g < CH)
                def _():
                    gather_op(j, b).wait()
                    pltpu.sync_copy(rows.at[b], acc.at[idxb.at[j, 1]],
                                    add=True)

                    @pl.when(g + 4 < CH)
                    def _():
                        idx_op(g + 4, j).start()

                    @pl.when(g + 2 < CH)
                    def _():
                        idx_op(g + 2, (j + 2) % 4).wait()
                        gather_op((j + 2) % 4, b).start()
            return carry

        lax.fori_loop(0, (CH + 3) // 4, outer, 0)
        plsc.subcore_barrier()
        pltpu.sync_copy(acc.at[pl.ds(s * SR, SR)],
                        agg_hbm.at[c, pl.ds(s * SR, SR)])
        if with_xgather:
            for k in range(XCH):
                pltpu.sync_copy(nos_hbm.at[s, k], nidx)
                pltpu.async_copy(h_hbm.at[c].at[nidx],
                                 rows.at[0].at[pl.ds(0, XC)], gs0).wait()
                pltpu.sync_copy(rows.at[0].at[pl.ds(0, XC)],
                                xp_hbm.at[c, pl.ds(s * SR + k * XC, XC)])

    return pl.kernel(body, out_type=out_type, mesh=mesh,
                     scratch_types=scratch)


def _grouped_conv(P, NT, NB, tile_deg, tile_valid, A, H, Wl_, Wr_, bl_):
    D = Wl_.shape[1]
    HD = D // 2

    def body(td_ref, tv_ref, a_ref, h_ref, wl_ref, wr_ref, b_ref, o_ref):
        i = pl.program_id(0)
        a = jnp.concatenate([a_ref[0], a_ref[1]], axis=1)
        hh = jnp.concatenate([h_ref[0], h_ref[1]], axis=1)
        out = (jnp.dot(a, wl_ref[0], preferred_element_type=jnp.float32)
               + jnp.dot(hh, wr_ref[0], preferred_element_type=jnp.float32))
        d = td_ref[i]
        bias = jnp.zeros((D,), jnp.float32)
        for dd in range(NB):
            bias = jnp.where(d == dd, b_ref[dd], bias)
        out = _leaky(out + bias[None, :])
        rid = lax.broadcasted_iota(jnp.int32, (T, 1), 0)
        out = jnp.where(rid < tv_ref[i], out, 0.0)
        o_ref[0] = out[:, :HD]
        o_ref[1] = out[:, HD:]

    grid_spec = pltpu.PrefetchScalarGridSpec(
        num_scalar_prefetch=2,
        grid=(NT,),
        in_specs=[
            pl.BlockSpec((2, T, HD), lambda i, td, tv: (0, i, 0)),
            pl.BlockSpec((2, T, HD), lambda i, td, tv: (0, i, 0)),
            pl.BlockSpec((1, D, D), lambda i, td, tv: (td[i], 0, 0)),
            pl.BlockSpec((1, D, D), lambda i, td, tv: (td[i], 0, 0)),
            pl.BlockSpec((NB, D), lambda i, td, tv: (0, 0)),
        ],
        out_specs=pl.BlockSpec((2, T, HD), lambda i, td, tv: (0, i, 0)),
    )
    return pl.pallas_call(
        body, grid_spec=grid_spec,
        out_shape=jax.ShapeDtypeStruct((2, P, HD), jnp.float32),
    )(tile_deg, tile_valid, A, H, Wl_, Wr_, bl_)


def _pool(P, NT, H, batch3d):
    def body(h_ref, b_ref, o_ref):
        i = pl.program_id(0)

        @pl.when(i == 0)
        def _():
            o_ref[...] = jnp.zeros_like(o_ref)

        bt = b_ref[0, 0]
        gid = lax.broadcasted_iota(jnp.int32, (NG, T), 0)
        oh = (gid == bt[None, :]).astype(jnp.float32)
        hh = jnp.concatenate([h_ref[0], h_ref[1]], axis=1)
        o_ref[...] += jnp.dot(oh, hh, preferred_element_type=jnp.float32)

    return pl.pallas_call(
        body,
        grid=(NT,),
        in_specs=[
            pl.BlockSpec((2, T, 128), lambda i: (0, i, 0)),
            pl.BlockSpec((1, 1, T), lambda i: (i, 0, 0)),
        ],
        out_specs=pl.BlockSpec((NG, 256), lambda i: (0, 0)),
        out_shape=jax.ShapeDtypeStruct((NG, 256), jnp.float32),
    )(H, batch3d)


def _classifier(P, NT, NC, h1, h2, h3, batch3d, pool, C1_W, C1_b2, CW, Cb,
                F_Wp, F_bp):
    def body(h1_ref, h2_ref, h3_ref, b_ref, p_ref, c1w_ref, c1b_ref,
             cw_ref, cb_ref, fw_ref, fb_ref, o_ref):
        bt = b_ref[0, 0]
        gid = lax.broadcasted_iota(jnp.int32, (T, NG), 1)
        oh = (gid == bt[:, None]).astype(jnp.float32)
        hp = jnp.dot(oh, p_ref[...], preferred_element_type=jnp.float32)
        hcat = jnp.concatenate(
            [jnp.concatenate([h1_ref[0], h1_ref[1]], axis=1),
             jnp.concatenate([h2_ref[0], h2_ref[1]], axis=1),
             jnp.concatenate([h3_ref[0], h3_ref[1]], axis=1),
             hp], axis=1)
        z = jnp.dot(hcat, c1w_ref[...],
                    preferred_element_type=jnp.float32) + c1b_ref[...]
        for l in range(NC):
            z = _leaky(jnp.dot(z, cw_ref[l],
                               preferred_element_type=jnp.float32)
                       + cb_ref[l][None, :])
        y = jnp.dot(z, fw_ref[...], preferred_element_type=jnp.float32)
        o_ref[...] = jax.nn.sigmoid(y + fb_ref[...])

    DC = C1_W.shape[1]
    return pl.pallas_call(
        body,
        grid=(NT,),
        in_specs=[
            pl.BlockSpec((2, T, 128), lambda i: (0, i, 0)),
            pl.BlockSpec((2, T, 128), lambda i: (0, i, 0)),
            pl.BlockSpec((2, T, 128), lambda i: (0, i, 0)),
            pl.BlockSpec((1, 1, T), lambda i: (i, 0, 0)),
            pl.BlockSpec((NG, 256), lambda i: (0, 0)),
            pl.BlockSpec(C1_W.shape, lambda i: (0, 0)),
            pl.BlockSpec((1, DC), lambda i: (0, 0)),
            pl.BlockSpec(CW.shape, lambda i: (0, 0, 0)),
            pl.BlockSpec(Cb.shape, lambda i: (0, 0)),
            pl.BlockSpec(F_Wp.shape, lambda i: (0, 0)),
            pl.BlockSpec((1, 128), lambda i: (0, 0)),
        ],
        out_specs=pl.BlockSpec((T, 128), lambda i: (i, 0)),
        out_shape=jax.ShapeDtypeStruct((P, 128), jnp.float32),
    )(h1, h2, h3, batch3d, pool, C1_W, C1_b2, CW, Cb, F_Wp, F_bp)


def _routing(deg, NB, N, P, NT):
    counts = jnp.bincount(deg, length=NB).astype(jnp.int32)
    padded = ((counts + T - 1) // T) * T
    z1 = jnp.zeros((1,), jnp.int32)
    pstart = jnp.concatenate([z1, jnp.cumsum(padded)])[:NB]
    sstart = jnp.concatenate([z1, jnp.cumsum(counts)])[:NB]
    perm = jnp.argsort(deg, stable=True).astype(jnp.int32)

    slots = jnp.arange(P, dtype=jnp.int32)
    b = (jnp.searchsorted(pstart, slots, side="right") - 1).astype(jnp.int32)
    off = slots - pstart[b]
    valid = off < counts[b]
    node_of_slot = jnp.where(
        valid,
        lax.optimization_barrier(perm[jnp.clip(sstart[b] + off, 0, N - 1)]),
        0).astype(jnp.int32)
    scat = jnp.where(valid, node_of_slot, N)
    slot_of_node = jnp.zeros((N,), jnp.int32).at[scat].set(slots, mode="drop")

    tstart = jnp.arange(NT, dtype=jnp.int32) * T
    tb = (jnp.searchsorted(pstart, tstart, side="right") - 1).astype(jnp.int32)
    tile_deg = tb
    tile_valid = jnp.clip(counts[tb] - (tstart - pstart[tb]), 0, T).astype(
        jnp.int32)
    return node_of_slot, slot_of_node, tile_deg, tile_valid


def kernel(x, edge_index, edge_attr, batch, Wl1, bl1, Wr1, Wl, bl, Wr,
           C1_W, C1_b, CW, Cb, F_W, F_b):
    N, D = x.shape
    E = edge_index.shape[1]
    NB = Wl1.shape[0]
    NCONV = Wl.shape[0]
    NC = CW.shape[0]
    DC = C1_W.shape[1]

    NT = -(-(N + NB * (T - 1)) // T)
    P = NT * T
    src = edge_index[0]
    dst = edge_index[1]

    deg = jnp.minimum(jnp.bincount(dst, length=N), NB - 1).astype(jnp.int32)
    node_of_slot, slot_of_node, tile_deg, tile_valid = _routing(
        deg, NB, N, P, NT)

    EP = -(-E // (NSUB * ECH)) * (NSUB * ECH)
    CH = EP // (NSUB * ECH)
    padn = EP - E

    @compute_on("tpu_sparsecore")
    @jax.jit
    def _relabel(tab, a, b):
        return tab[a], tab[b]

    dst_slot, src_slot = _relabel(slot_of_node, dst, src)
    trash = jnp.full((padn,), P - 1, jnp.int32)
    src_l1 = jnp.concatenate([src.astype(jnp.int32),
                              jnp.zeros((padn,), jnp.int32)]).reshape(
                                  NSUB, CH, ECH)
    src_ln = jnp.concatenate([src_slot, trash]).reshape(NSUB, CH, ECH)
    dst_e = jnp.concatenate([dst_slot, trash]).reshape(NSUB, CH, ECH)
    idx_l1 = jnp.stack([src_l1, dst_e], axis=2)
    idx_ln = jnp.stack([src_ln, dst_e], axis=2)

    SR = P // NSUB
    XC = 120
    XCH = SR // XC
    nos3 = node_of_slot.reshape(NSUB, XCH, XC)

    batch3d = lax.optimization_barrier(
        batch[node_of_slot].astype(jnp.int32)).reshape(NT, 1, T)
    x_split = jnp.stack([x[:, :128], x[:, 128:]])
    zerosP = jnp.zeros((P, 128), jnp.float32)

    sc1 = _make_sc_segsum(N, P, CH, True, XC, XCH)
    scn = _make_sc_segsum(P, P, CH, False)

    agg1, xperm = sc1(x_split, idx_l1, zerosP, nos3)
    h = _grouped_conv(P, NT, NB, tile_deg, tile_valid, agg1, xperm,
                      Wl1, Wr1, bl1)
    hs = []
    for l in range(NCONV):
        (aggl,) = scn(h, idx_ln, zerosP)
        h = _grouped_conv(P, NT, NB, tile_deg, tile_valid, aggl, h,
                          Wl[l], Wr[l], bl[l])
        hs.append(h)

    pool = _pool(P, NT, hs[-1], batch3d)

    F_Wp = jnp.pad(F_W, ((0, 0), (0, 127)))
    F_bp = jnp.pad(F_b.reshape(1, 1), ((0, 0), (0, 127)))
    cls = _classifier(P, NT, NC, hs[0], hs[1], hs[2], batch3d, pool,
                      C1_W, C1_b.reshape(1, DC), CW, Cb, F_Wp, F_bp)
    return lax.optimization_barrier(cls[:, 0][slot_of_node])[:, None]


# speedup vs baseline: 2.3589x; 1.0055x over previous
import functools

import jax
import jax.numpy as jnp
from jax import lax
from jax.experimental import pallas as pl
from jax.experimental.pallas import tpu as pltpu
from jax.experimental.pallas import tpu_sc as plsc
from jax.experimental.compute_on import compute_on

T = 128
LEAK = 0.01
NG = 64
NSUB = 16
ECH = 128


def _leaky(x):
    return jnp.where(x >= 0, x, LEAK * x)


def _make_sc_segsum(NH, P, CH, with_xgather, XC=0, XCH=0):
    SR = P // NSUB
    mesh = plsc.VectorSubcoreMesh(core_axis_name="c", subcore_axis_name="s")
    out_type = [jax.ShapeDtypeStruct((2, P, 128), jnp.float32)]
    if with_xgather:
        out_type.append(jax.ShapeDtypeStruct((2, P, 128), jnp.float32))
    scratch = [
        pltpu.VMEM((4, 2, ECH), jnp.int32),
        pltpu.VMEM((2, ECH, 128), jnp.float32),
        pltpu.VMEM_SHARED((P, 128), jnp.float32),
        pltpu.SemaphoreType.DMA,
        pltpu.SemaphoreType.DMA,
        pltpu.SemaphoreType.DMA,
        pltpu.SemaphoreType.DMA,
        pltpu.SemaphoreType.DMA,
        pltpu.SemaphoreType.DMA,
    ]
    if with_xgather:
        scratch.append(pltpu.VMEM((XC,), jnp.int32))

    def body(h_hbm, idx_hbm, zeros_hbm, *rest):
        if with_xgather:
            (nos_hbm, agg_hbm, xp_hbm, idxb, rows, acc,
             is0, is1, is2, is3, gs0, gs1, nidx) = rest
        else:
            agg_hbm, idxb, rows, acc, is0, is1, is2, is3, gs0, gs1 = rest
        isems = (is0, is1, is2, is3)
        gsems = (gs0, gs1)
        c = lax.axis_index("c")
        s = lax.axis_index("s")
        pltpu.sync_copy(zeros_hbm.at[pl.ds(s * SR, SR)],
                        acc.at[pl.ds(s * SR, SR)])

        def idx_op(g, j):
            return pltpu.make_async_copy(idx_hbm.at[s, g], idxb.at[j],
                                         isems[j])

        def gather_op(j, b):
            return pltpu.make_async_copy(
                h_hbm.at[c].at[idxb.at[j, 0]], rows.at[b], gsems[b])

        for j in range(4):
            idx_op(j, j).start()
        for j in range(2):
            idx_op(j, j).wait()
            gather_op(j, j).start()
        plsc.subcore_barrier()

        def outer(i, carry):
            g0 = 4 * i
            for j in range(4):
                g = g0 + j
                b = j % 2

                @pl.when(g < CH)
                def _():
                    gather_op(j, b).wait()
                    pltpu.sync_copy(rows.at[b], acc.at[idxb.at[j, 1]],
                                    add=True)

                    @pl.when(g + 4 < CH)
                    def _():
                        idx_op(g + 4, j).start()

                    @pl.when(g + 2 < CH)
                    def _():
                        idx_op(g + 2, (j + 2) % 4).wait()
                        gather_op((j + 2) % 4, b).start()
            return carry

        lax.fori_loop(0, (CH + 3) // 4, outer, 0)
        plsc.subcore_barrier()
        pltpu.sync_copy(acc.at[pl.ds(s * SR, SR)],
                        agg_hbm.at[c, pl.ds(s * SR, SR)])
        if with_xgather:
            for k in range(XCH):
                pltpu.sync_copy(nos_hbm.at[s, k], nidx)
                pltpu.async_copy(h_hbm.at[c].at[nidx],
                                 rows.at[0].at[pl.ds(0, XC)], gs0).wait()
                pltpu.sync_copy(rows.at[0].at[pl.ds(0, XC)],
                                xp_hbm.at[c, pl.ds(s * SR + k * XC, XC)])

    return pl.kernel(body, out_type=out_type, mesh=mesh,
                     scratch_types=scratch)


def _grouped_conv(P, NT, NB, tile_deg, tile_valid, A, H, Wl_, Wr_, bl_):
    D = Wl_.shape[1]
    HD = D // 2

    def body(td_ref, tv_ref, a_ref, h_ref, wl_ref, wr_ref, b_ref, o_ref):
        i = pl.program_id(0)
        a = jnp.concatenate([a_ref[0], a_ref[1]], axis=1)
        hh = jnp.concatenate([h_ref[0], h_ref[1]], axis=1)
        out = (jnp.dot(a, wl_ref[0], preferred_element_type=jnp.float32)
               + jnp.dot(hh, wr_ref[0], preferred_element_type=jnp.float32))
        d = td_ref[i]
        bias = jnp.zeros((D,), jnp.float32)
        for dd in range(NB):
            bias = jnp.where(d == dd, b_ref[dd], bias)
        out = _leaky(out + bias[None, :])
        rid = lax.broadcasted_iota(jnp.int32, (T, 1), 0)
        out = jnp.where(rid < tv_ref[i], out, 0.0)
        o_ref[0] = out[:, :HD]
        o_ref[1] = out[:, HD:]

    grid_spec = pltpu.PrefetchScalarGridSpec(
        num_scalar_prefetch=2,
        grid=(NT,),
        in_specs=[
            pl.BlockSpec((2, T, HD), lambda i, td, tv: (0, i, 0)),
            pl.BlockSpec((2, T, HD), lambda i, td, tv: (0, i, 0)),
            pl.BlockSpec((1, D, D), lambda i, td, tv: (td[i], 0, 0)),
            pl.BlockSpec((1, D, D), lambda i, td, tv: (td[i], 0, 0)),
            pl.BlockSpec((NB, D), lambda i, td, tv: (0, 0)),
        ],
        out_specs=pl.BlockSpec((2, T, HD), lambda i, td, tv: (0, i, 0)),
    )
    return pl.pallas_call(
        body, grid_spec=grid_spec,
        out_shape=jax.ShapeDtypeStruct((2, P, HD), jnp.float32),
    )(tile_deg, tile_valid, A, H, Wl_, Wr_, bl_)


def _conv_right(P, NT, NB, tile_deg, H, Wr_, bl_):
    D = Wr_.shape[1]
    HD = D // 2

    def body(td_ref, h_ref, wr_ref, b_ref, o_ref):
        i = pl.program_id(0)
        hh = jnp.concatenate([h_ref[0], h_ref[1]], axis=1)
        out = jnp.dot(hh, wr_ref[0], preferred_element_type=jnp.float32)
        d = td_ref[i]
        bias = jnp.zeros((D,), jnp.float32)
        for dd in range(NB):
            bias = jnp.where(d == dd, b_ref[dd], bias)
        out = out + bias[None, :]
        o_ref[0] = out[:, :HD]
        o_ref[1] = out[:, HD:]

    grid_spec = pltpu.PrefetchScalarGridSpec(
        num_scalar_prefetch=1,
        grid=(NT,),
        in_specs=[
            pl.BlockSpec((2, T, HD), lambda i, td: (0, i, 0)),
            pl.BlockSpec((1, D, D), lambda i, td: (td[i], 0, 0)),
            pl.BlockSpec((NB, D), lambda i, td: (0, 0)),
        ],
        out_specs=pl.BlockSpec((2, T, HD), lambda i, td: (0, i, 0)),
    )
    return pl.pallas_call(
        body, grid_spec=grid_spec,
        out_shape=jax.ShapeDtypeStruct((2, P, HD), jnp.float32),
    )(tile_deg, H, Wr_, bl_)


def _conv_left(P, NT, tile_deg, tile_valid, A, HR, Wl_):
    D = Wl_.shape[1]
    HD = D // 2

    def body(td_ref, tv_ref, a_ref, hr_ref, wl_ref, o_ref):
        i = pl.program_id(0)
        a = jnp.concatenate([a_ref[0], a_ref[1]], axis=1)
        hr = jnp.concatenate([hr_ref[0], hr_ref[1]], axis=1)
        out = jnp.dot(a, wl_ref[0], preferred_element_type=jnp.float32) + hr
        out = _leaky(out)
        rid = lax.broadcasted_iota(jnp.int32, (T, 1), 0)
        out = jnp.where(rid < tv_ref[i], out, 0.0)
        o_ref[0] = out[:, :HD]
        o_ref[1] = out[:, HD:]

    grid_spec = pltpu.PrefetchScalarGridSpec(
        num_scalar_prefetch=2,
        grid=(NT,),
        in_specs=[
            pl.BlockSpec((2, T, HD), lambda i, td, tv: (0, i, 0)),
            pl.BlockSpec((2, T, HD), lambda i, td, tv: (0, i, 0)),
            pl.BlockSpec((1, D, D), lambda i, td, tv: (td[i], 0, 0)),
        ],
        out_specs=pl.BlockSpec((2, T, HD), lambda i, td, tv: (0, i, 0)),
    )
    return pl.pallas_call(
        body, grid_spec=grid_spec,
        out_shape=jax.ShapeDtypeStruct((2, P, HD), jnp.float32),
    )(tile_deg, tile_valid, A, HR, Wl_)


def _pool(P, NT, H, batch3d):
    def body(h_ref, b_ref, o_ref):
        i = pl.program_id(0)

        @pl.when(i == 0)
        def _():
            o_ref[...] = jnp.zeros_like(o_ref)

        bt = b_ref[0, 0]
        gid = lax.broadcasted_iota(jnp.int32, (NG, T), 0)
        oh = (gid == bt[None, :]).astype(jnp.float32)
        hh = jnp.concatenate([h_ref[0], h_ref[1]], axis=1)
        o_ref[...] += jnp.dot(oh, hh, preferred_element_type=jnp.float32)

    return pl.pallas_call(
        body,
        grid=(NT,),
        in_specs=[
            pl.BlockSpec((2, T, 128), lambda i: (0, i, 0)),
            pl.BlockSpec((1, 1, T), lambda i: (i, 0, 0)),
        ],
        out_specs=pl.BlockSpec((NG, 256), lambda i: (0, 0)),
        out_shape=jax.ShapeDtypeStruct((NG, 256), jnp.float32),
    )(H, batch3d)


def _classifier(P, NT, NC, h1, h2, h3, batch3d, pool, C1_W, C1_b2, CW, Cb,
                F_Wp, F_bp):
    def body(h1_ref, h2_ref, h3_ref, b_ref, p_ref, c1w_ref, c1b_ref,
             cw_ref, cb_ref, fw_ref, fb_ref, o_ref):
        bt = b_ref[0, 0]
        gid = lax.broadcasted_iota(jnp.int32, (T, NG), 1)
        oh = (gid == bt[:, None]).astype(jnp.float32)
        hp = jnp.dot(oh, p_ref[...], preferred_element_type=jnp.float32)
        hcat = jnp.concatenate(
            [jnp.concatenate([h1_ref[0], h1_ref[1]], axis=1),
             jnp.concatenate([h2_ref[0], h2_ref[1]], axis=1),
             jnp.concatenate([h3_ref[0], h3_ref[1]], axis=1),
             hp], axis=1)
        z = jnp.dot(hcat, c1w_ref[...],
                    preferred_element_type=jnp.float32) + c1b_ref[...]
        for l in range(NC):
            z = _leaky(jnp.dot(z, cw_ref[l],
                               preferred_element_type=jnp.float32)
                       + cb_ref[l][None, :])
        y = jnp.dot(z, fw_ref[...], preferred_element_type=jnp.float32)
        o_ref[...] = jax.nn.sigmoid(y + fb_ref[...])

    DC = C1_W.shape[1]
    return pl.pallas_call(
        body,
        grid=(NT,),
        in_specs=[
            pl.BlockSpec((2, T, 128), lambda i: (0, i, 0)),
            pl.BlockSpec((2, T, 128), lambda i: (0, i, 0)),
            pl.BlockSpec((2, T, 128), lambda i: (0, i, 0)),
            pl.BlockSpec((1, 1, T), lambda i: (i, 0, 0)),
            pl.BlockSpec((NG, 256), lambda i: (0, 0)),
            pl.BlockSpec(C1_W.shape, lambda i: (0, 0)),
            pl.BlockSpec((1, DC), lambda i: (0, 0)),
            pl.BlockSpec(CW.shape, lambda i: (0, 0, 0)),
            pl.BlockSpec(Cb.shape, lambda i: (0, 0)),
            pl.BlockSpec(F_Wp.shape, lambda i: (0, 0)),
            pl.BlockSpec((1, 128), lambda i: (0, 0)),
        ],
        out_specs=pl.BlockSpec((T, 128), lambda i: (i, 0)),
        out_shape=jax.ShapeDtypeStruct((P, 128), jnp.float32),
    )(h1, h2, h3, batch3d, pool, C1_W, C1_b2, CW, Cb, F_Wp, F_bp)


def _routing(deg, NB, N, P, NT):
    counts = jnp.bincount(deg, length=NB).astype(jnp.int32)
    padded = ((counts + T - 1) // T) * T
    z1 = jnp.zeros((1,), jnp.int32)
    pstart = jnp.concatenate([z1, jnp.cumsum(padded)])[:NB]
    sstart = jnp.concatenate([z1, jnp.cumsum(counts)])[:NB]
    perm = jnp.argsort(deg, stable=True).astype(jnp.int32)

    slots = jnp.arange(P, dtype=jnp.int32)
    b = (jnp.searchsorted(pstart, slots, side="right") - 1).astype(jnp.int32)
    off = slots - pstart[b]
    valid = off < counts[b]
    node_of_slot = jnp.where(
        valid,
        lax.optimization_barrier(perm[jnp.clip(sstart[b] + off, 0, N - 1)]),
        0).astype(jnp.int32)
    scat = jnp.where(valid, node_of_slot, N)
    slot_of_node = jnp.zeros((N,), jnp.int32).at[scat].set(slots, mode="drop")

    tstart = jnp.arange(NT, dtype=jnp.int32) * T
    tb = (jnp.searchsorted(pstart, tstart, side="right") - 1).astype(jnp.int32)
    tile_deg = tb
    tile_valid = jnp.clip(counts[tb] - (tstart - pstart[tb]), 0, T).astype(
        jnp.int32)
    return node_of_slot, slot_of_node, tile_deg, tile_valid


def kernel(x, edge_index, edge_attr, batch, Wl1, bl1, Wr1, Wl, bl, Wr,
           C1_W, C1_b, CW, Cb, F_W, F_b):
    N, D = x.shape
    E = edge_index.shape[1]
    NB = Wl1.shape[0]
    NCONV = Wl.shape[0]
    NC = CW.shape[0]
    DC = C1_W.shape[1]

    NT = -(-(N + NB * (T - 1)) // T)
    P = NT * T
    src = edge_index[0]
    dst = edge_index[1]

    deg = jnp.minimum(jnp.bincount(dst, length=N), NB - 1).astype(jnp.int32)
    node_of_slot, slot_of_node, tile_deg, tile_valid = _routing(
        deg, NB, N, P, NT)

    EP = -(-E // (NSUB * ECH)) * (NSUB * ECH)
    CH = EP // (NSUB * ECH)
    padn = EP - E

    @compute_on("tpu_sparsecore")
    @jax.jit
    def _relabel(tab, a, b):
        return tab[a], tab[b]

    dst_slot, src_slot = _relabel(slot_of_node, dst, src)
    trash = jnp.full((padn,), P - 1, jnp.int32)
    src_l1 = jnp.concatenate([src.astype(jnp.int32),
                              jnp.zeros((padn,), jnp.int32)]).reshape(
                                  NSUB, CH, ECH)
    src_ln = jnp.concatenate([src_slot, trash]).reshape(NSUB, CH, ECH)
    dst_e = jnp.concatenate([dst_slot, trash]).reshape(NSUB, CH, ECH)
    idx_l1 = jnp.stack([src_l1, dst_e], axis=2)
    idx_ln = jnp.stack([src_ln, dst_e], axis=2)

    SR = P // NSUB
    XC = 120
    XCH = SR // XC
    nos3 = node_of_slot.reshape(NSUB, XCH, XC)

    batch3d = lax.optimization_barrier(
        batch[node_of_slot].astype(jnp.int32)).reshape(NT, 1, T)
    x_split = jnp.stack([x[:, :128], x[:, 128:]])
    zerosP = jnp.zeros((P, 128), jnp.float32)

    sc1 = _make_sc_segsum(N, P, CH, True, XC, XCH)
    scn = _make_sc_segsum(P, P, CH, False)

    agg1, xperm = sc1(x_split, idx_l1, zerosP, nos3)
    h = _grouped_conv(P, NT, NB, tile_deg, tile_valid, agg1, xperm,
                      Wl1, Wr1, bl1)
    hs = []
    for l in range(NCONV):
        (aggl,) = scn(h, idx_ln, zerosP)
        hr = _conv_right(P, NT, NB, tile_deg, h, Wr[l], bl[l])
        h = _conv_left(P, NT, tile_deg, tile_valid, aggl, hr, Wl[l])
        hs.append(h)

    pool = _pool(P, NT, hs[-1], batch3d)

    F_Wp = jnp.pad(F_W, ((0, 0), (0, 127)))
    F_bp = jnp.pad(F_b.reshape(1, 1), ((0, 0), (0, 127)))
    cls = _classifier(P, NT, NC, hs[0], hs[1], hs[2], batch3d, pool,
                      C1_W, C1_b.reshape(1, DC), CW, Cb, F_Wp, F_bp)
    return lax.optimization_barrier(cls[:, 0][slot_of_node])[:, None]


# speedup vs baseline: 2.3592x; 1.0001x over previous
import functools

import jax
import jax.numpy as jnp
from jax import lax
from jax.experimental import pallas as pl
from jax.experimental.pallas import tpu as pltpu
from jax.experimental.pallas import tpu_sc as plsc
from jax.experimental.compute_on import compute_on

T = 128
LEAK = 0.01
NG = 64
NSUB = 16
ECH = 128


def _leaky(x):
    return jnp.where(x >= 0, x, LEAK * x)


def _make_sc_segsum(NH, P, CH, with_xgather, XC=0, XCH=0):
    SR = P // NSUB
    mesh = plsc.VectorSubcoreMesh(core_axis_name="c", subcore_axis_name="s")
    out_type = [jax.ShapeDtypeStruct((2, P, 128), jnp.float32)]
    if with_xgather:
        out_type.append(jax.ShapeDtypeStruct((2, P, 128), jnp.float32))
    scratch = [
        pltpu.VMEM((4, 2, ECH), jnp.int32),
        pltpu.VMEM((2, ECH, 128), jnp.float32),
        pltpu.VMEM_SHARED((P, 128), jnp.float32),
        pltpu.SemaphoreType.DMA,
        pltpu.SemaphoreType.DMA,
        pltpu.SemaphoreType.DMA,
        pltpu.SemaphoreType.DMA,
        pltpu.SemaphoreType.DMA,
        pltpu.SemaphoreType.DMA,
    ]
    if with_xgather:
        scratch.append(pltpu.VMEM((XCH, XC), jnp.int32))

    def body(h_hbm, idx_hbm, zeros_hbm, *rest):
        if with_xgather:
            (nos_hbm, agg_hbm, xp_hbm, idxb, rows, acc,
             is0, is1, is2, is3, gs0, gs1, nidx) = rest
        else:
            agg_hbm, idxb, rows, acc, is0, is1, is2, is3, gs0, gs1 = rest
        isems = (is0, is1, is2, is3)
        gsems = (gs0, gs1)
        c = lax.axis_index("c")
        s = lax.axis_index("s")
        pltpu.sync_copy(zeros_hbm.at[pl.ds(s * SR, SR)],
                        acc.at[pl.ds(s * SR, SR)])

        def idx_op(g, j):
            return pltpu.make_async_copy(idx_hbm.at[s, g], idxb.at[j],
                                         isems[j])

        def gather_op(j, b):
            return pltpu.make_async_copy(
                h_hbm.at[c].at[idxb.at[j, 0]], rows.at[b], gsems[b])

        for j in range(4):
            idx_op(j, j).start()
        for j in range(2):
            idx_op(j, j).wait()
            gather_op(j, j).start()
        plsc.subcore_barrier()

        def outer(i, carry):
            g0 = 4 * i
            for j in range(4):
                g = g0 + j
                b = j % 2

                @pl.when(g < CH)
                def _():
                    gather_op(j, b).wait()
                    pltpu.sync_copy(rows.at[b], acc.at[idxb.at[j, 1]],
                                    add=True)

                    @pl.when(g + 4 < CH)
                    def _():
                        idx_op(g + 4, j).start()

                    @pl.when(g + 2 < CH)
                    def _():
                        idx_op(g + 2, (j + 2) % 4).wait()
                        gather_op((j + 2) % 4, b).start()
            return carry

        lax.fori_loop(0, (CH + 3) // 4, outer, 0)
        plsc.subcore_barrier()
        pltpu.sync_copy(acc.at[pl.ds(s * SR, SR)],
                        agg_hbm.at[c, pl.ds(s * SR, SR)])
        if with_xgather:
            pltpu.sync_copy(nos_hbm.at[s], nidx)

            def xg(k):
                return pltpu.make_async_copy(
                    h_hbm.at[c].at[nidx.at[k]],
                    rows.at[k % 2].at[pl.ds(0, XC)], gsems[k % 2])

            xg(0).start()
            for k in range(XCH):
                xg(k).wait()
                if k + 1 < XCH:
                    xg(k + 1).start()
                pltpu.sync_copy(rows.at[k % 2].at[pl.ds(0, XC)],
                                xp_hbm.at[c, pl.ds(s * SR + k * XC, XC)])

    return pl.kernel(body, out_type=out_type, mesh=mesh,
                     scratch_types=scratch)


def _grouped_conv(P, NT, NB, tile_deg, tile_valid, A, H, Wl_, Wr_, bl_):
    D = Wl_.shape[1]
    HD = D // 2

    def body(td_ref, tv_ref, a_ref, h_ref, wl_ref, wr_ref, b_ref, o_ref):
        i = pl.program_id(0)
        a = jnp.concatenate([a_ref[0], a_ref[1]], axis=1)
        hh = jnp.concatenate([h_ref[0], h_ref[1]], axis=1)
        out = (jnp.dot(a, wl_ref[0], preferred_element_type=jnp.float32)
               + jnp.dot(hh, wr_ref[0], preferred_element_type=jnp.float32))
        d = td_ref[i]
        bias = jnp.zeros((D,), jnp.float32)
        for dd in range(NB):
            bias = jnp.where(d == dd, b_ref[dd], bias)
        out = _leaky(out + bias[None, :])
        rid = lax.broadcasted_iota(jnp.int32, (T, 1), 0)
        out = jnp.where(rid < tv_ref[i], out, 0.0)
        o_ref[0] = out[:, :HD]
        o_ref[1] = out[:, HD:]

    grid_spec = pltpu.PrefetchScalarGridSpec(
        num_scalar_prefetch=2,
        grid=(NT,),
        in_specs=[
            pl.BlockSpec((2, T, HD), lambda i, td, tv: (0, i, 0)),
            pl.BlockSpec((2, T, HD), lambda i, td, tv: (0, i, 0)),
            pl.BlockSpec((1, D, D), lambda i, td, tv: (td[i], 0, 0)),
            pl.BlockSpec((1, D, D), lambda i, td, tv: (td[i], 0, 0)),
            pl.BlockSpec((NB, D), lambda i, td, tv: (0, 0)),
        ],
        out_specs=pl.BlockSpec((2, T, HD), lambda i, td, tv: (0, i, 0)),
    )
    return pl.pallas_call(
        body, grid_spec=grid_spec,
        out_shape=jax.ShapeDtypeStruct((2, P, HD), jnp.float32),
    )(tile_deg, tile_valid, A, H, Wl_, Wr_, bl_)


def _conv_right(P, NT, NB, tile_deg, H, Wr_, bl_):
    D = Wr_.shape[1]
    HD = D // 2

    def body(td_ref, h_ref, wr_ref, b_ref, o_ref):
        i = pl.program_id(0)
        hh = jnp.concatenate([h_ref[0], h_ref[1]], axis=1)
        out = jnp.dot(hh, wr_ref[0], preferred_element_type=jnp.float32)
        d = td_ref[i]
        bias = jnp.zeros((D,), jnp.float32)
        for dd in range(NB):
            bias = jnp.where(d == dd, b_ref[dd], bias)
        out = out + bias[None, :]
        o_ref[0] = out[:, :HD]
        o_ref[1] = out[:, HD:]

    grid_spec = pltpu.PrefetchScalarGridSpec(
        num_scalar_prefetch=1,
        grid=(NT,),
        in_specs=[
            pl.BlockSpec((2, T, HD), lambda i, td: (0, i, 0)),
            pl.BlockSpec((1, D, D), lambda i, td: (td[i], 0, 0)),
            pl.BlockSpec((NB, D), lambda i, td: (0, 0)),
        ],
        out_specs=pl.BlockSpec((2, T, HD), lambda i, td: (0, i, 0)),
    )
    return pl.pallas_call(
        body, grid_spec=grid_spec,
        out_shape=jax.ShapeDtypeStruct((2, P, HD), jnp.float32),
    )(tile_deg, H, Wr_, bl_)


def _conv_left(P, NT, tile_deg, tile_valid, A, HR, Wl_):
    D = Wl_.shape[1]
    HD = D // 2

    def body(td_ref, tv_ref, a_ref, hr_ref, wl_ref, o_ref):
        i = pl.program_id(0)
        a = jnp.concatenate([a_ref[0], a_ref[1]], axis=1)
        hr = jnp.concatenate([hr_ref[0], hr_ref[1]], axis=1)
        out = jnp.dot(a, wl_ref[0], preferred_element_type=jnp.float32) + hr
        out = _leaky(out)
        rid = lax.broadcasted_iota(jnp.int32, (T, 1), 0)
        out = jnp.where(rid < tv_ref[i], out, 0.0)
        o_ref[0] = out[:, :HD]
        o_ref[1] = out[:, HD:]

    grid_spec = pltpu.PrefetchScalarGridSpec(
        num_scalar_prefetch=2,
        grid=(NT,),
        in_specs=[
            pl.BlockSpec((2, T, HD), lambda i, td, tv: (0, i, 0)),
            pl.BlockSpec((2, T, HD), lambda i, td, tv: (0, i, 0)),
            pl.BlockSpec((1, D, D), lambda i, td, tv: (td[i], 0, 0)),
        ],
        out_specs=pl.BlockSpec((2, T, HD), lambda i, td, tv: (0, i, 0)),
    )
    return pl.pallas_call(
        body, grid_spec=grid_spec,
        out_shape=jax.ShapeDtypeStruct((2, P, HD), jnp.float32),
    )(tile_deg, tile_valid, A, HR, Wl_)


def _pool(P, NT, H, batch3d):
    def body(h_ref, b_ref, o_ref):
        i = pl.program_id(0)

        @pl.when(i == 0)
        def _():
            o_ref[...] = jnp.zeros_like(o_ref)

        bt = b_ref[0, 0]
        gid = lax.broadcasted_iota(jnp.int32, (NG, T), 0)
        oh = (gid == bt[None, :]).astype(jnp.float32)
        hh = jnp.concatenate([h_ref[0], h_ref[1]], axis=1)
        o_ref[...] += jnp.dot(oh, hh, preferred_element_type=jnp.float32)

    return pl.pallas_call(
        body,
        grid=(NT,),
        in_specs=[
            pl.BlockSpec((2, T, 128), lambda i: (0, i, 0)),
            pl.BlockSpec((1, 1, T), lambda i: (i, 0, 0)),
        ],
        out_specs=pl.BlockSpec((NG, 256), lambda i: (0, 0)),
        out_shape=jax.ShapeDtypeStruct((NG, 256), jnp.float32),
    )(H, batch3d)


def _classifier(P, NT, NC, h1, h2, h3, batch3d, pool, C1_W, C1_b2, CW, Cb,
                F_Wp, F_bp):
    def body(h1_ref, h2_ref, h3_ref, b_ref, p_ref, c1w_ref, c1b_ref,
             cw_ref, cb_ref, fw_ref, fb_ref, o_ref):
        bt = b_ref[0, 0]
        gid = lax.broadcasted_iota(jnp.int32, (T, NG), 1)
        oh = (gid == bt[:, None]).astype(jnp.float32)
        hp = jnp.dot(oh, p_ref[...], preferred_element_type=jnp.float32)
        hcat = jnp.concatenate(
            [jnp.concatenate([h1_ref[0], h1_ref[1]], axis=1),
             jnp.concatenate([h2_ref[0], h2_ref[1]], axis=1),
             jnp.concatenate([h3_ref[0], h3_ref[1]], axis=1),
             hp], axis=1)
        z = jnp.dot(hcat, c1w_ref[...],
                    preferred_element_type=jnp.float32) + c1b_ref[...]
        for l in range(NC):
            z = _leaky(jnp.dot(z, cw_ref[l],
                               preferred_element_type=jnp.float32)
                       + cb_ref[l][None, :])
        y = jnp.dot(z, fw_ref[...], preferred_element_type=jnp.float32)
        o_ref[...] = jax.nn.sigmoid(y + fb_ref[...])

    DC = C1_W.shape[1]
    return pl.pallas_call(
        body,
        grid=(NT,),
        in_specs=[
            pl.BlockSpec((2, T, 128), lambda i: (0, i, 0)),
            pl.BlockSpec((2, T, 128), lambda i: (0, i, 0)),
            pl.BlockSpec((2, T, 128), lambda i: (0, i, 0)),
            pl.BlockSpec((1, 1, T), lambda i: (i, 0, 0)),
            pl.BlockSpec((NG, 256), lambda i: (0, 0)),
            pl.BlockSpec(C1_W.shape, lambda i: (0, 0)),
            pl.BlockSpec((1, DC), lambda i: (0, 0)),
            pl.BlockSpec(CW.shape, lambda i: (0, 0, 0)),
            pl.BlockSpec(Cb.shape, lambda i: (0, 0)),
            pl.BlockSpec(F_Wp.shape, lambda i: (0, 0)),
            pl.BlockSpec((1, 128), lambda i: (0, 0)),
        ],
        out_specs=pl.BlockSpec((T, 128), lambda i: (i, 0)),
        out_shape=jax.ShapeDtypeStruct((P, 128), jnp.float32),
    )(h1, h2, h3, batch3d, pool, C1_W, C1_b2, CW, Cb, F_Wp, F_bp)


def _routing(deg, NB, N, P, NT):
    counts = jnp.bincount(deg, length=NB).astype(jnp.int32)
    padded = ((counts + T - 1) // T) * T
    z1 = jnp.zeros((1,), jnp.int32)
    pstart = jnp.concatenate([z1, jnp.cumsum(padded)])[:NB]
    sstart = jnp.concatenate([z1, jnp.cumsum(counts)])[:NB]
    perm = jnp.argsort(deg, stable=True).astype(jnp.int32)

    slots = jnp.arange(P, dtype=jnp.int32)
    b = (jnp.searchsorted(pstart, slots, side="right") - 1).astype(jnp.int32)
    off = slots - pstart[b]
    valid = off < counts[b]
    node_of_slot = jnp.where(
        valid,
        lax.optimization_barrier(perm[jnp.clip(sstart[b] + off, 0, N - 1)]),
        0).astype(jnp.int32)
    scat = jnp.where(valid, node_of_slot, N)
    slot_of_node = jnp.zeros((N,), jnp.int32).at[scat].set(slots, mode="drop")

    tstart = jnp.arange(NT, dtype=jnp.int32) * T
    tb = (jnp.searchsorted(pstart, tstart, side="right") - 1).astype(jnp.int32)
    tile_deg = tb
    tile_valid = jnp.clip(counts[tb] - (tstart - pstart[tb]), 0, T).astype(
        jnp.int32)
    return node_of_slot, slot_of_node, tile_deg, tile_valid


def kernel(x, edge_index, edge_attr, batch, Wl1, bl1, Wr1, Wl, bl, Wr,
           C1_W, C1_b, CW, Cb, F_W, F_b):
    N, D = x.shape
    E = edge_index.shape[1]
    NB = Wl1.shape[0]
    NCONV = Wl.shape[0]
    NC = CW.shape[0]
    DC = C1_W.shape[1]

    NT = -(-(N + NB * (T - 1)) // T)
    P = NT * T
    src = edge_index[0]
    dst = edge_index[1]

    deg = jnp.minimum(jnp.bincount(dst, length=N), NB - 1).astype(jnp.int32)
    node_of_slot, slot_of_node, tile_deg, tile_valid = _routing(
        deg, NB, N, P, NT)

    EP = -(-E // (NSUB * ECH)) * (NSUB * ECH)
    CH = EP // (NSUB * ECH)
    padn = EP - E

    @compute_on("tpu_sparsecore")
    @jax.jit
    def _relabel(tab, a, b):
        return tab[a], tab[b]

    dst_slot, src_slot = _relabel(slot_of_node, dst, src)
    trash = jnp.full((padn,), P - 1, jnp.int32)
    src_l1 = jnp.concatenate([src.astype(jnp.int32),
                              jnp.zeros((padn,), jnp.int32)]).reshape(
                                  NSUB, CH, ECH)
    src_ln = jnp.concatenate([src_slot, trash]).reshape(NSUB, CH, ECH)
    dst_e = jnp.concatenate([dst_slot, trash]).reshape(NSUB, CH, ECH)
    idx_l1 = jnp.stack([src_l1, dst_e], axis=2)
    idx_ln = jnp.stack([src_ln, dst_e], axis=2)

    SR = P // NSUB
    XC = 120
    XCH = SR // XC
    nos3 = node_of_slot.reshape(NSUB, XCH, XC)

    batch3d = lax.optimization_barrier(
        batch[node_of_slot].astype(jnp.int32)).reshape(NT, 1, T)
    x_split = jnp.stack([x[:, :128], x[:, 128:]])
    zerosP = jnp.zeros((P, 128), jnp.float32)

    sc1 = _make_sc_segsum(N, P, CH, True, XC, XCH)
    scn = _make_sc_segsum(P, P, CH, False)

    agg1, xperm = sc1(x_split, idx_l1, zerosP, nos3)
    h = _grouped_conv(P, NT, NB, tile_deg, tile_valid, agg1, xperm,
                      Wl1, Wr1, bl1)
    hs = []
    for l in range(NCONV):
        (aggl,) = scn(h, idx_ln, zerosP)
        hr = _conv_right(P, NT, NB, tile_deg, h, Wr[l], bl[l])
        h = _conv_left(P, NT, tile_deg, tile_valid, aggl, hr, Wl[l])
        hs.append(h)

    pool = _pool(P, NT, hs[-1], batch3d)

    F_Wp = jnp.pad(F_W, ((0, 0), (0, 127)))
    F_bp = jnp.pad(F_b.reshape(1, 1), ((0, 0), (0, 127)))
    cls = _classifier(P, NT, NC, hs[0], hs[1], hs[2], batch3d, pool,
                      C1_W, C1_b.reshape(1, DC), CW, Cb, F_Wp, F_bp)
    return lax.optimization_barrier(cls[:, 0][slot_of_node])[:, None]
